# Initial kernel scaffold; baseline (speedup 1.0000x reference)
import jax
import jax.numpy as jnp
from jax.experimental import pallas as pl
from jax.experimental.pallas import tpu as pltpu

_B, _T, _D = 4, 4096, 2048
_W = 128
_K = 1024
_TT = 512


def _stats_body(a_ref, p_ref, dst_ref, dch_ref):
    a = a_ref[...]
    p = p_ref[...]
    inv_d = jnp.float32(1.0 / _D)
    dst_ref[...] = jnp.sum(a * a, axis=-1) * inv_d
    d = a - p
    dch_ref[...] = jnp.sum(d * d, axis=-1) * inv_d


def _prefix_sum(x):
    n = x.shape[1]
    s = 1
    while s < n:
        z = jnp.zeros((x.shape[0], s), x.dtype)
        x = x + jnp.concatenate([z, x[:, : n - s]], axis=1)
        s *= 2
    return x


def _route_body(scal_ref, dst_ref, dch_ref, g_ref, m_ref):
    c_ce = scal_ref[0]
    m_cu = scal_ref[1]
    bce = scal_ref[2]
    bcu = scal_ref[3]
    d_st = dst_ref[...]
    d_ch = dch_ref[...]

    ce = d_st - (d_ch - c_ce)
    csum = _prefix_sum(d_st)
    shifted = jnp.concatenate(
        [jnp.zeros((_B, _W), jnp.float32), csum[:, : _T - _W]], axis=1)
    wsum = csum - shifted
    pos = jax.lax.broadcasted_iota(jnp.float32, (_B, _T), 1)
    counts = jnp.minimum(pos + 1.0, jnp.float32(_W))
    cu = d_st - m_cu * (wsum / counts)

    s_ce = 1.0 / (1.0 + jnp.exp(-bce * ce))
    s_cu = 1.0 / (1.0 + jnp.exp(-bcu * cu))
    g = s_ce + s_cu - s_ce * s_cu
    g_ref[...] = g

    bits = jax.lax.bitcast_convert_type(g, jnp.int32)
    lo = jnp.zeros((_B, 1), jnp.int32)
    hi = jnp.full((_B, 1), 0x3F800001, jnp.int32)

    def body(_, carry):
        lo, hi = carry
        mid = lo + (hi - lo) // 2
        cnt = jnp.sum((bits >= mid).astype(jnp.int32), axis=1, keepdims=True)
        ge = cnt >= _K
        return jnp.where(ge, mid, lo), jnp.where(ge, hi, mid)

    lo, _ = jax.lax.fori_loop(0, 31, body, (lo, hi))
    tau = lo
    gt = bits > tau
    eq = bits == tau
    cnt_gt = jnp.sum(gt.astype(jnp.int32), axis=1, keepdims=True)
    need = _K - cnt_gt
    eq_rank = _prefix_sum(eq.astype(jnp.int32))
    mask = gt | (eq & (eq_rank <= need))
    m_ref[...] = mask.astype(jnp.float32)


def kernel(actual_residual, predicted_residual, raw_o_ce, raw_m_cu, beta_ce, beta_cu):
    d_st, d_ch = pl.pallas_call(
        _stats_body,
        grid=(_B, _T // _TT),
        in_specs=[
            pl.BlockSpec((1, _TT, _D), lambda b, t: (b, t, 0)),
            pl.BlockSpec((1, _TT, _D), lambda b, t: (b, t, 0)),
        ],
        out_specs=[
            pl.BlockSpec((1, _TT), lambda b, t: (b, t)),
            pl.BlockSpec((1, _TT), lambda b, t: (b, t)),
        ],
        out_shape=[
            jax.ShapeDtypeStruct((_B, _T), jnp.float32),
            jax.ShapeDtypeStruct((_B, _T), jnp.float32),
        ],
    )(actual_residual, predicted_residual)

    o_ce_pos = jax.nn.softplus(jnp.asarray(raw_o_ce, jnp.float32))
    m_cu_pos = jax.nn.softplus(jnp.asarray(raw_m_cu, jnp.float32))
    scal = jnp.stack([
        jnp.log(o_ce_pos + 1e-10),
        m_cu_pos,
        jnp.asarray(beta_ce, jnp.float32),
        jnp.asarray(beta_cu, jnp.float32),
    ])

    g, mask = pl.pallas_call(
        _route_body,
        in_specs=[
            pl.BlockSpec(memory_space=pltpu.SMEM),
            pl.BlockSpec(memory_space=pltpu.VMEM),
            pl.BlockSpec(memory_space=pltpu.VMEM),
        ],
        out_specs=[
            pl.BlockSpec(memory_space=pltpu.VMEM),
            pl.BlockSpec(memory_space=pltpu.VMEM),
        ],
        out_shape=[
            jax.ShapeDtypeStruct((_B, _T), jnp.float32),
            jax.ShapeDtypeStruct((_B, _T), jnp.float32),
        ],
    )(scal, d_st, d_ch)
    return (g, mask)


# speedup vs baseline: 1.2583x; 1.2583x over previous
import jax
import jax.numpy as jnp
from jax.experimental import pallas as pl
from jax.experimental.pallas import tpu as pltpu

_B, _T, _D = 4, 4096, 2048
_W = 128
_K = 1024
_TT = 256


def _stats_body(a_ref, p_ref, dst_ref, dch_ref):
    a = a_ref[...]
    p = p_ref[...]
    inv_d = jnp.float32(1.0 / _D)
    dst_ref[...] = jnp.sum(a * a, axis=-1) * inv_d
    d = a - p
    dch_ref[...] = jnp.sum(d * d, axis=-1) * inv_d


def _prefix_sum(x):
    n = x.shape[1]
    s = 1
    while s < n:
        z = jnp.zeros((x.shape[0], s), x.dtype)
        x = x + jnp.concatenate([z, x[:, : n - s]], axis=1)
        s *= 2
    return x


def _route_body(scal_ref, dst_ref, dch_ref, g_ref, m_ref):
    c_ce = scal_ref[0]
    m_cu = scal_ref[1]
    bce = scal_ref[2]
    bcu = scal_ref[3]
    d_st = dst_ref[...]
    d_ch = dch_ref[...]

    ce = d_st - (d_ch - c_ce)
    csum = _prefix_sum(d_st)
    shifted = jnp.concatenate(
        [jnp.zeros((_B, _W), jnp.float32), csum[:, : _T - _W]], axis=1)
    wsum = csum - shifted
    pos = jax.lax.broadcasted_iota(jnp.int32, (_B, _T), 1).astype(jnp.float32)
    counts = jnp.minimum(pos + 1.0, jnp.float32(_W))
    cu = d_st - m_cu * (wsum / counts)

    s_ce = 1.0 / (1.0 + jnp.exp(-bce * ce))
    s_cu = 1.0 / (1.0 + jnp.exp(-bcu * cu))
    g = s_ce + s_cu - s_ce * s_cu
    g_ref[...] = g

    bits = jax.lax.bitcast_convert_type(g, jnp.int32)
    lo = jnp.zeros((_B, 1), jnp.int32)
    hi = jnp.full((_B, 1), 0x3F800001, jnp.int32)

    def body(_, carry):
        lo, hi = carry
        mid = lo + (hi - lo) // 2
        cnt = jnp.sum((bits >= mid).astype(jnp.int32), axis=1, keepdims=True)
        ge = cnt >= _K
        return jnp.where(ge, mid, lo), jnp.where(ge, hi, mid)

    lo, _ = jax.lax.fori_loop(0, 31, body, (lo, hi))
    tau = lo
    gt = bits > tau
    eq = bits == tau
    cnt_gt = jnp.sum(gt.astype(jnp.int32), axis=1, keepdims=True)
    need = _K - cnt_gt
    eq_rank = _prefix_sum(eq.astype(jnp.int32))
    mask = gt | (eq & (eq_rank <= need))
    m_ref[...] = mask.astype(jnp.float32)


def kernel(actual_residual, predicted_residual, raw_o_ce, raw_m_cu, beta_ce, beta_cu):
    d_st, d_ch = pl.pallas_call(
        _stats_body,
        grid=(_T // _TT,),
        in_specs=[
            pl.BlockSpec((_B, _TT, _D), lambda t: (0, t, 0)),
            pl.BlockSpec((_B, _TT, _D), lambda t: (0, t, 0)),
        ],
        out_specs=[
            pl.BlockSpec((_B, _TT), lambda t: (0, t)),
            pl.BlockSpec((_B, _TT), lambda t: (0, t)),
        ],
        out_shape=[
            jax.ShapeDtypeStruct((_B, _T), jnp.float32),
            jax.ShapeDtypeStruct((_B, _T), jnp.float32),
        ],
    )(actual_residual, predicted_residual)

    o_ce_pos = jax.nn.softplus(jnp.asarray(raw_o_ce, jnp.float32))
    m_cu_pos = jax.nn.softplus(jnp.asarray(raw_m_cu, jnp.float32))
    scal = jnp.stack([
        jnp.log(o_ce_pos + 1e-10),
        m_cu_pos,
        jnp.asarray(beta_ce, jnp.float32),
        jnp.asarray(beta_cu, jnp.float32),
    ])

    g, mask = pl.pallas_call(
        _route_body,
        in_specs=[
            pl.BlockSpec(memory_space=pltpu.SMEM),
            pl.BlockSpec(memory_space=pltpu.VMEM),
            pl.BlockSpec(memory_space=pltpu.VMEM),
        ],
        out_specs=[
            pl.BlockSpec(memory_space=pltpu.VMEM),
            pl.BlockSpec(memory_space=pltpu.VMEM),
        ],
        out_shape=[
            jax.ShapeDtypeStruct((_B, _T), jnp.float32),
            jax.ShapeDtypeStruct((_B, _T), jnp.float32),
        ],
    )(scal, d_st, d_ch)
    return (g, mask)


# speedup vs baseline: 1.2798x; 1.0171x over previous
import jax
import jax.numpy as jnp
from jax.experimental import pallas as pl
from jax.experimental.pallas import tpu as pltpu

_B, _T, _D = 4, 4096, 2048
_W = 128
_K = 1024
_TT = 256
_NT = _T // _TT


def _prefix_sum(x):
    n = x.shape[1]
    s = 1
    while s < n:
        z = jnp.zeros((x.shape[0], s), x.dtype)
        x = x + jnp.concatenate([z, x[:, : n - s]], axis=1)
        s *= 2
    return x


def _routing(scal_ref, d_st, d_ch, g_ref, m_ref):
    c_ce = scal_ref[0]
    m_cu = scal_ref[1]
    bce = scal_ref[2]
    bcu = scal_ref[3]

    ce = d_st - (d_ch - c_ce)
    csum = _prefix_sum(d_st)
    shifted = jnp.concatenate(
        [jnp.zeros((_B, _W), jnp.float32), csum[:, : _T - _W]], axis=1)
    wsum = csum - shifted
    pos = jax.lax.broadcasted_iota(jnp.int32, (_B, _T), 1).astype(jnp.float32)
    counts = jnp.minimum(pos + 1.0, jnp.float32(_W))
    cu = d_st - m_cu * (wsum / counts)

    s_ce = 1.0 / (1.0 + jnp.exp(-bce * ce))
    s_cu = 1.0 / (1.0 + jnp.exp(-bcu * cu))
    g = s_ce + s_cu - s_ce * s_cu
    g_ref[...] = g

    bits = jax.lax.bitcast_convert_type(g, jnp.int32)
    lo = jnp.zeros((_B, 1), jnp.int32)
    hi = jnp.full((_B, 1), 0x3F800001, jnp.int32)

    def body(_, carry):
        lo, hi = carry
        mid = lo + (hi - lo) // 2
        cnt = jnp.sum((bits >= mid).astype(jnp.int32), axis=1, keepdims=True)
        ge = cnt >= _K
        return jnp.where(ge, mid, lo), jnp.where(ge, hi, mid)

    lo, _ = jax.lax.fori_loop(0, 31, body, (lo, hi))
    tau = lo
    gt = bits > tau
    eq = bits == tau
    cnt_gt = jnp.sum(gt.astype(jnp.int32), axis=1, keepdims=True)
    need = _K - cnt_gt
    eq_rank = _prefix_sum(eq.astype(jnp.int32))
    mask = gt | (eq & (eq_rank <= need))
    m_ref[...] = mask.astype(jnp.float32)


def _fused_body(scal_ref, a_ref, p_ref, g_ref, m_ref, dst_s, dch_s):
    t = pl.program_id(0)
    a = a_ref[...]
    p = p_ref[...]
    inv_d = jnp.float32(1.0 / _D)
    d = a - p
    dst_s[:, pl.ds(t * _TT, _TT)] = jnp.sum(a * a, axis=-1) * inv_d
    dch_s[:, pl.ds(t * _TT, _TT)] = jnp.sum(d * d, axis=-1) * inv_d

    @pl.when(t == _NT - 1)
    def _():
        _routing(scal_ref, dst_s[...], dch_s[...], g_ref, m_ref)


def kernel(actual_residual, predicted_residual, raw_o_ce, raw_m_cu, beta_ce, beta_cu):
    o_ce_pos = jax.nn.softplus(jnp.asarray(raw_o_ce, jnp.float32))
    m_cu_pos = jax.nn.softplus(jnp.asarray(raw_m_cu, jnp.float32))
    scal = jnp.stack([
        jnp.log(o_ce_pos + 1e-10),
        m_cu_pos,
        jnp.asarray(beta_ce, jnp.float32),
        jnp.asarray(beta_cu, jnp.float32),
    ])

    g, mask = pl.pallas_call(
        _fused_body,
        grid=(_NT,),
        in_specs=[
            pl.BlockSpec(memory_space=pltpu.SMEM),
            pl.BlockSpec((_B, _TT, _D), lambda t: (0, t, 0)),
            pl.BlockSpec((_B, _TT, _D), lambda t: (0, t, 0)),
        ],
        out_specs=[
            pl.BlockSpec((_B, _T), lambda t: (0, 0)),
            pl.BlockSpec((_B, _T), lambda t: (0, 0)),
        ],
        out_shape=[
            jax.ShapeDtypeStruct((_B, _T), jnp.float32),
            jax.ShapeDtypeStruct((_B, _T), jnp.float32),
        ],
        scratch_shapes=[
            pltpu.VMEM((_B, _T), jnp.float32),
            pltpu.VMEM((_B, _T), jnp.float32),
        ],
    )(scal, actual_residual, predicted_residual)
    return (g, mask)
